# baseline (speedup 1.0000x reference)
import functools

import jax
import jax.numpy as jnp
from jax import lax
from jax.experimental import pallas as pl
from jax.experimental.pallas import tpu as pltpu
from jax.experimental.pallas import tpu_sc as plsc

N = 10000
E = 320000
H = 128
VOCAB = 250
B = 64
L = 3

NP = 10112
VP = 256
NTILES = 32
CH = 128
KCH = 80
KC0 = 128
KC1 = 32
EP = NTILES * KCH * CH
RPT = NP // 16
RB = NP // 8
BNS = (1.0 + 1e-5) ** -0.5


def _embed_body(ids_ref, emb_ref, out_ref):
    ids = ids_ref[0, 0, :]
    cols = lax.broadcasted_iota(jnp.int32, (128, VP), 1)
    oh = (ids[:, None] == cols).astype(jnp.float32)
    out_ref[...] = jax.nn.relu(
        jnp.dot(oh, emb_ref[...], preferred_element_type=jnp.float32))


def _embed(ids3, embp):
    return pl.pallas_call(
        _embed_body,
        grid=(NP // 128,),
        in_specs=[
            pl.BlockSpec((1, 1, 128), lambda i: (i, 0, 0)),
            pl.BlockSpec((VP, H), lambda i: (0, 0)),
        ],
        out_specs=pl.BlockSpec((128, H), lambda i: (i, 0)),
        out_shape=jax.ShapeDtypeStruct((NP, H), jnp.float32),
    )(ids3, embp)


DEGC = 2048


def _deg_body(dst_ref, out_ref):
    step = pl.program_id(0)

    @pl.when(step == 0)
    def _():
        out_ref[...] = jnp.zeros((128, NP // 128), jnp.float32)

    d = dst_ref[0, 0, :]
    hi = d // 128
    lo = d - hi * 128
    lns = lax.broadcasted_iota(jnp.int32, (128, DEGC), 0)
    bmT = (lns == lo[None, :]).astype(jnp.float32)
    rws = lax.broadcasted_iota(jnp.int32, (DEGC, NP // 128), 1)
    aT = (rws == hi[:, None]).astype(jnp.float32)
    out_ref[...] += jnp.dot(bmT, aT, preferred_element_type=jnp.float32)


def _deg(dst3):
    return pl.pallas_call(
        _deg_body,
        grid=(EP // DEGC,),
        in_specs=[pl.BlockSpec((1, 1, DEGC), lambda i: (i, 0, 0))],
        out_specs=pl.BlockSpec((128, NP // 128), lambda i: (0, 0)),
        out_shape=jax.ShapeDtypeStruct((128, NP // 128), jnp.float32),
    )(dst3)


def _dinv_body(deg_ref, out_ref):
    col = deg_ref[...]
    out_ref[...] = jnp.broadcast_to(lax.rsqrt(col + 1.0), (128, H))


def _dinv(deg_col):
    return pl.pallas_call(
        _dinv_body,
        grid=(NP // 128,),
        in_specs=[pl.BlockSpec((128, 1), lambda i: (i, 0))],
        out_specs=pl.BlockSpec((128, H), lambda i: (i, 0)),
        out_shape=jax.ShapeDtypeStruct((NP, H), jnp.float32),
    )(deg_col)


def _mm0_body(x_ref, dinv_ref, w_ref, out_ref):
    out_ref[...] = dinv_ref[...] * jnp.dot(
        x_ref[...], w_ref[...], preferred_element_type=jnp.float32)


def _mm0(x0, dinvH, W):
    blk = lambda i: (i, 0)
    return pl.pallas_call(
        _mm0_body,
        grid=(NP // RB,),
        in_specs=[
            pl.BlockSpec((RB, H), blk),
            pl.BlockSpec((RB, H), blk),
            pl.BlockSpec((H, H), lambda i: (0, 0)),
        ],
        out_specs=pl.BlockSpec((RB, H), blk),
        out_shape=jax.ShapeDtypeStruct((NP, H), jnp.float32),
    )(x0, dinvH, W)


def _mmf_body(p_ref, g_ref, dinv_ref, b_ref, w_ref, out_ref):
    dinv = dinv_ref[...]
    x = jax.nn.relu(dinv * (p_ref[0] + p_ref[1] + g_ref[...]) + b_ref[...])
    out_ref[...] = dinv * jnp.dot(
        x, w_ref[...], preferred_element_type=jnp.float32)


def _mmf(parts, g, dinvH, b2d, W):
    blk = lambda i: (i, 0)
    return pl.pallas_call(
        _mmf_body,
        grid=(NP // RB,),
        in_specs=[
            pl.BlockSpec((2, RB, H), lambda i: (0, i, 0)),
            pl.BlockSpec((RB, H), blk),
            pl.BlockSpec((RB, H), blk),
            pl.BlockSpec((1, H), lambda i: (0, 0)),
            pl.BlockSpec((H, H), lambda i: (0, 0)),
        ],
        out_specs=pl.BlockSpec((RB, H), blk),
        out_shape=jax.ShapeDtypeStruct((NP, H), jnp.float32),
    )(parts, g, dinvH, b2d, W)


def _pool_body(p_ref, g_ref, dinv_ref, b_ref, bat_ref,
               l0w_ref, l0b_ref, gam_ref, bet_ref, l1w_ref, l1b_ref,
               out_ref):
    step = pl.program_id(0)

    @pl.when(step == 0)
    def _():
        out_ref[...] = jnp.zeros((B, H), jnp.float32)

    x = jax.nn.relu(
        dinv_ref[...] * (p_ref[0] + p_ref[1] + g_ref[...]) + b_ref[...])
    bid = bat_ref[0, 0, :]
    rows = lax.broadcasted_iota(jnp.int32, (B, RB), 0)
    m = (rows == bid[None, :]).astype(jnp.float32)
    out_ref[...] += jnp.dot(m, x, preferred_element_type=jnp.float32)

    @pl.when(step == pl.num_programs(0) - 1)
    def _():
        pooled = out_ref[...]
        h = jnp.dot(pooled, l0w_ref[...],
                    preferred_element_type=jnp.float32) + l0b_ref[...]
        h = h * BNS * gam_ref[...] + bet_ref[...]
        h = jax.nn.relu(h)
        lg = jnp.dot(h, l1w_ref[...],
                     preferred_element_type=jnp.float32) + l1b_ref[...]
        mx = jnp.max(lg, axis=1, keepdims=True)
        e = jnp.exp(lg - mx)
        out_ref[...] = e / jnp.sum(e, axis=1, keepdims=True)


def _pool(parts, g, dinvH, b2d, bat3, l0w, l0b, gam, bet, l1wp, l1bp):
    blk = lambda i: (i, 0)
    one = lambda i: (0, 0)
    return pl.pallas_call(
        _pool_body,
        grid=(NP // RB,),
        in_specs=[
            pl.BlockSpec((2, RB, H), lambda i: (0, i, 0)),
            pl.BlockSpec((RB, H), blk),
            pl.BlockSpec((RB, H), blk),
            pl.BlockSpec((1, H), one),
            pl.BlockSpec((1, 1, RB), lambda i: (i, 0, 0)),
            pl.BlockSpec((H, H), one),
            pl.BlockSpec((1, H), one),
            pl.BlockSpec((1, H), one),
            pl.BlockSpec((1, H), one),
            pl.BlockSpec((H, H), one),
            pl.BlockSpec((1, H), one),
        ],
        out_specs=pl.BlockSpec((B, H), lambda i: (0, 0)),
        out_shape=jax.ShapeDtypeStruct((B, H), jnp.float32),
    )(parts, g, dinvH, b2d, bat3, l0w, l0b, gam, bet, l1wp, l1bp)


@functools.lru_cache(maxsize=None)
def _mesh():
    return plsc.VectorSubcoreMesh(core_axis_name="c", subcore_axis_name="s")


def _msg_sc(g, src2, dst2):
    def body(g_hbm, src_hbm, dst_hbm, out_hbm,
             sidx, didx, rows, shared, sr0, sr1, si0, si1):
        c = lax.axis_index("c")
        s = lax.axis_index("s")
        base = s * (KC0 + KC1) + c * KC0
        nloop = (KC0 - 2) // 2 - ((KC0 - KC1) // 2) * c
        row0 = s * RPT
        sems_r = (sr0, sr1)
        sems_i = (si0, si1)

        def fetch_idx(j, b):
            pltpu.async_copy(src_hbm.at[base + j], sidx.at[b], sems_i[b])
            pltpu.async_copy(dst_hbm.at[base + j], didx.at[b], sems_i[b])

        def wait_idx(b):
            pltpu.make_async_copy(src_hbm.at[base], sidx.at[b],
                                  sems_i[b]).wait()
            pltpu.make_async_copy(dst_hbm.at[base], didx.at[b],
                                  sems_i[b]).wait()

        def start_gather(b):
            pltpu.async_copy(g_hbm.at[sidx.at[b]], rows.at[b], sems_r[b])

        def wait_gather(b):
            pltpu.make_async_copy(g_hbm.at[sidx.at[b]], rows.at[b],
                                  sems_r[b]).wait()

        def scatter(b):
            pltpu.sync_copy(rows.at[b], shared.at[didx.at[b]], add=True)

        def fill_z(i, _):
            for b in range(2):
                for k in range(8):
                    rows[b, i, pl.ds(k * 16, 16)] = jnp.zeros((16,),
                                                              jnp.float32)
            return _
        lax.fori_loop(0, CH, fill_z, None)

        for z in range(4):
            pltpu.sync_copy(rows.at[0], shared.at[pl.ds(row0 + z * CH, CH)])
        pltpu.sync_copy(rows.at[1, pl.ds(0, RPT - 4 * CH)],
                        shared.at[pl.ds(row0 + 4 * CH, RPT - 4 * CH)])
        plsc.subcore_barrier()

        fetch_idx(0, 0)
        fetch_idx(1, 1)
        wait_idx(0)
        start_gather(0)

        def loop(i, _):
            j2 = i * 2
            for b in range(2):
                wait_idx(1 - b)
                wait_gather(b)
                start_gather(1 - b)
                scatter(b)
                fetch_idx(j2 + b + 2, b)
            return _
        lax.fori_loop(0, nloop, loop, None)

        wait_idx(1)
        wait_gather(0)
        start_gather(1)
        scatter(0)
        wait_gather(1)
        scatter(1)

        plsc.subcore_barrier()
        pltpu.sync_copy(shared.at[pl.ds(row0, RPT)],
                        out_hbm.at[c, pl.ds(row0, RPT)])

    return pl.kernel(
        body,
        out_type=jax.ShapeDtypeStruct((2, NP, H), jnp.float32),
        mesh=_mesh(),
        scratch_types=[
            pltpu.VMEM((2, CH), jnp.int32),
            pltpu.VMEM((2, CH), jnp.int32),
            pltpu.VMEM((2, CH, H), jnp.float32),
            pltpu.VMEM_SHARED((NP, H), jnp.float32),
            pltpu.SemaphoreType.DMA,
            pltpu.SemaphoreType.DMA,
            pltpu.SemaphoreType.DMA,
            pltpu.SemaphoreType.DMA,
        ],
    )(g, src2, dst2)


def kernel(x_p_id, edge_index_p, x_p_batch, emb0, conv_W, conv_b,
           lin0_W, lin0_b, bn_gamma, bn_beta, lin1_W, lin1_b):
    f32 = jnp.float32
    ids3 = jnp.pad(x_p_id.astype(jnp.int32), (0, NP - N)).reshape(NP // 128, 1, 128)
    embp = jnp.pad(emb0, ((0, VP - VOCAB), (0, 0)))
    src2 = jnp.pad(edge_index_p[0].astype(jnp.int32), (0, EP - E),
                   constant_values=N).reshape(NTILES * KCH, CH)
    dst2 = jnp.pad(edge_index_p[1].astype(jnp.int32), (0, EP - E),
                   constant_values=N).reshape(NTILES * KCH, CH)
    dst_deg = jnp.pad(edge_index_p[1].astype(jnp.int32), (0, EP - E),
                      constant_values=N).reshape(EP // 2048, 1, 2048)
    bat3 = jnp.pad(x_p_batch.astype(jnp.int32), (0, NP - N),
                   constant_values=B + 1).reshape(NP // RB, 1, RB)
    b2 = [conv_b[i].reshape(1, H) for i in range(L)]
    l0b = lin0_b.reshape(1, H)
    gam = bn_gamma.reshape(1, H)
    bet = bn_beta.reshape(1, H)
    l1wp = jnp.pad(lin1_W, ((0, 0), (0, H - 2)))
    l1bp = jnp.pad(lin1_b, (0, H - 2), constant_values=-1e30).reshape(1, H)

    x0 = _embed(ids3, embp)
    deg2 = _deg(dst_deg)
    dinvH = _dinv(deg2.T.reshape(NP, 1))

    g = _mm0(x0, dinvH, conv_W[0])
    parts = _msg_sc(g, src2, dst2)
    for i in (1, 2):
        g = _mmf(parts, g, dinvH, b2[i - 1], conv_W[i])
        parts = _msg_sc(g, src2, dst2)

    out = _pool(parts, g, dinvH, b2[2], bat3, lin0_W, l0b, gam, bet,
                l1wp, l1bp)
    return out[:, :2].astype(f32)


# speedup vs baseline: 1.0836x; 1.0836x over previous
import functools

import jax
import jax.numpy as jnp
from jax import lax
from jax.experimental import pallas as pl
from jax.experimental.pallas import tpu as pltpu
from jax.experimental.pallas import tpu_sc as plsc

N = 10000
E = 320000
H = 128
VOCAB = 250
B = 64
L = 3

NP = 10112
VP = 256
NTILES = 32
CH = 128
KCH = 80
KC0 = 120
KC1 = 40
EP = NTILES * KCH * CH
RPT = NP // 16
RB = NP // 8
BNS = (1.0 + 1e-5) ** -0.5


def _embed_body(ids_ref, emb_ref, out_ref):
    ids = ids_ref[0, 0, :]
    cols = lax.broadcasted_iota(jnp.int32, (128, VP), 1)
    oh = (ids[:, None] == cols).astype(jnp.float32)
    out_ref[...] = jax.nn.relu(
        jnp.dot(oh, emb_ref[...], preferred_element_type=jnp.float32))


def _embed(ids3, embp):
    return pl.pallas_call(
        _embed_body,
        grid=(NP // 128,),
        in_specs=[
            pl.BlockSpec((1, 1, 128), lambda i: (i, 0, 0)),
            pl.BlockSpec((VP, H), lambda i: (0, 0)),
        ],
        out_specs=pl.BlockSpec((128, H), lambda i: (i, 0)),
        out_shape=jax.ShapeDtypeStruct((NP, H), jnp.float32),
    )(ids3, embp)


DEGC = 2048


def _deg_body(dst_ref, out_ref):
    step = pl.program_id(0)

    @pl.when(step == 0)
    def _():
        out_ref[...] = jnp.zeros((128, NP // 128), jnp.float32)

    d = dst_ref[0, 0, :]
    hi = d // 128
    lo = d - hi * 128
    lns = lax.broadcasted_iota(jnp.int32, (128, DEGC), 0)
    bmT = (lns == lo[None, :]).astype(jnp.float32)
    rws = lax.broadcasted_iota(jnp.int32, (DEGC, NP // 128), 1)
    aT = (rws == hi[:, None]).astype(jnp.float32)
    out_ref[...] += jnp.dot(bmT, aT, preferred_element_type=jnp.float32)


def _deg(dst3):
    return pl.pallas_call(
        _deg_body,
        grid=(EP // DEGC,),
        in_specs=[pl.BlockSpec((1, 1, DEGC), lambda i: (i, 0, 0))],
        out_specs=pl.BlockSpec((128, NP // 128), lambda i: (0, 0)),
        out_shape=jax.ShapeDtypeStruct((128, NP // 128), jnp.float32),
    )(dst3)


def _dinv_body(deg_ref, out_ref):
    col = deg_ref[...]
    out_ref[...] = jnp.broadcast_to(lax.rsqrt(col + 1.0), (128, H))


def _dinv(deg_col):
    return pl.pallas_call(
        _dinv_body,
        grid=(NP // 128,),
        in_specs=[pl.BlockSpec((128, 1), lambda i: (i, 0))],
        out_specs=pl.BlockSpec((128, H), lambda i: (i, 0)),
        out_shape=jax.ShapeDtypeStruct((NP, H), jnp.float32),
    )(deg_col)


def _mm0_body(x_ref, dinv_ref, w_ref, out_ref):
    out_ref[...] = dinv_ref[...] * jnp.dot(
        x_ref[...], w_ref[...], preferred_element_type=jnp.float32)


def _mm0(x0, dinvH, W):
    blk = lambda i: (i, 0)
    return pl.pallas_call(
        _mm0_body,
        grid=(NP // RB,),
        in_specs=[
            pl.BlockSpec((RB, H), blk),
            pl.BlockSpec((RB, H), blk),
            pl.BlockSpec((H, H), lambda i: (0, 0)),
        ],
        out_specs=pl.BlockSpec((RB, H), blk),
        out_shape=jax.ShapeDtypeStruct((NP, H), jnp.float32),
    )(x0, dinvH, W)


def _mmf_body(p_ref, g_ref, dinv_ref, b_ref, w_ref, out_ref):
    dinv = dinv_ref[...]
    x = jax.nn.relu(dinv * (p_ref[0] + p_ref[1] + g_ref[...]) + b_ref[...])
    out_ref[...] = dinv * jnp.dot(
        x, w_ref[...], preferred_element_type=jnp.float32)


def _mmf(parts, g, dinvH, b2d, W):
    blk = lambda i: (i, 0)
    return pl.pallas_call(
        _mmf_body,
        grid=(NP // RB,),
        in_specs=[
            pl.BlockSpec((2, RB, H), lambda i: (0, i, 0)),
            pl.BlockSpec((RB, H), blk),
            pl.BlockSpec((RB, H), blk),
            pl.BlockSpec((1, H), lambda i: (0, 0)),
            pl.BlockSpec((H, H), lambda i: (0, 0)),
        ],
        out_specs=pl.BlockSpec((RB, H), blk),
        out_shape=jax.ShapeDtypeStruct((NP, H), jnp.float32),
    )(parts, g, dinvH, b2d, W)


def _pool_body(p_ref, g_ref, dinv_ref, b_ref, bat_ref,
               l0w_ref, l0b_ref, gam_ref, bet_ref, l1w_ref, l1b_ref,
               out_ref):
    step = pl.program_id(0)

    @pl.when(step == 0)
    def _():
        out_ref[...] = jnp.zeros((B, H), jnp.float32)

    x = jax.nn.relu(
        dinv_ref[...] * (p_ref[0] + p_ref[1] + g_ref[...]) + b_ref[...])
    bid = bat_ref[0, 0, :]
    rows = lax.broadcasted_iota(jnp.int32, (B, RB), 0)
    m = (rows == bid[None, :]).astype(jnp.float32)
    out_ref[...] += jnp.dot(m, x, preferred_element_type=jnp.float32)

    @pl.when(step == pl.num_programs(0) - 1)
    def _():
        pooled = out_ref[...]
        h = jnp.dot(pooled, l0w_ref[...],
                    preferred_element_type=jnp.float32) + l0b_ref[...]
        h = h * BNS * gam_ref[...] + bet_ref[...]
        h = jax.nn.relu(h)
        lg = jnp.dot(h, l1w_ref[...],
                     preferred_element_type=jnp.float32) + l1b_ref[...]
        mx = jnp.max(lg, axis=1, keepdims=True)
        e = jnp.exp(lg - mx)
        out_ref[...] = e / jnp.sum(e, axis=1, keepdims=True)


def _pool(parts, g, dinvH, b2d, bat3, l0w, l0b, gam, bet, l1wp, l1bp):
    blk = lambda i: (i, 0)
    one = lambda i: (0, 0)
    return pl.pallas_call(
        _pool_body,
        grid=(NP // RB,),
        in_specs=[
            pl.BlockSpec((2, RB, H), lambda i: (0, i, 0)),
            pl.BlockSpec((RB, H), blk),
            pl.BlockSpec((RB, H), blk),
            pl.BlockSpec((1, H), one),
            pl.BlockSpec((1, 1, RB), lambda i: (i, 0, 0)),
            pl.BlockSpec((H, H), one),
            pl.BlockSpec((1, H), one),
            pl.BlockSpec((1, H), one),
            pl.BlockSpec((1, H), one),
            pl.BlockSpec((H, H), one),
            pl.BlockSpec((1, H), one),
        ],
        out_specs=pl.BlockSpec((B, H), lambda i: (0, 0)),
        out_shape=jax.ShapeDtypeStruct((B, H), jnp.float32),
    )(parts, g, dinvH, b2d, bat3, l0w, l0b, gam, bet, l1wp, l1bp)


@functools.lru_cache(maxsize=None)
def _mesh():
    return plsc.VectorSubcoreMesh(core_axis_name="c", subcore_axis_name="s")


def _msg_sc(g, src2, dst2):
    def body(g_hbm, src_hbm, dst_hbm, out_hbm,
             sidx, didx, rows, shared, sr0, sr1, si0, si1):
        c = lax.axis_index("c")
        s = lax.axis_index("s")
        base = s * (KC0 + KC1) + c * KC0
        nloop = (KC0 - 2) // 2 - ((KC0 - KC1) // 2) * c
        row0 = s * RPT
        sems_r = (sr0, sr1)
        sems_i = (si0, si1)

        def fetch_idx(j, b):
            pltpu.async_copy(src_hbm.at[base + j], sidx.at[b], sems_i[b])
            pltpu.async_copy(dst_hbm.at[base + j], didx.at[b], sems_i[b])

        def wait_idx(b):
            pltpu.make_async_copy(src_hbm.at[base], sidx.at[b],
                                  sems_i[b]).wait()
            pltpu.make_async_copy(dst_hbm.at[base], didx.at[b],
                                  sems_i[b]).wait()

        def start_gather(b):
            pltpu.async_copy(g_hbm.at[sidx.at[b]], rows.at[b], sems_r[b])

        def wait_gather(b):
            pltpu.make_async_copy(g_hbm.at[sidx.at[b]], rows.at[b],
                                  sems_r[b]).wait()

        def scatter(b):
            pltpu.sync_copy(rows.at[b], shared.at[didx.at[b]], add=True)

        def fill_z(i, _):
            for b in range(2):
                for k in range(8):
                    rows[b, i, pl.ds(k * 16, 16)] = jnp.zeros((16,),
                                                              jnp.float32)
            return _
        lax.fori_loop(0, CH, fill_z, None)

        for z in range(4):
            pltpu.sync_copy(rows.at[0], shared.at[pl.ds(row0 + z * CH, CH)])
        pltpu.sync_copy(rows.at[1, pl.ds(0, RPT - 4 * CH)],
                        shared.at[pl.ds(row0 + 4 * CH, RPT - 4 * CH)])
        plsc.subcore_barrier()

        fetch_idx(0, 0)
        fetch_idx(1, 1)
        wait_idx(0)
        start_gather(0)

        def loop(i, _):
            j2 = i * 2
            for b in range(2):
                wait_idx(1 - b)
                wait_gather(b)
                start_gather(1 - b)
                scatter(b)
                fetch_idx(j2 + b + 2, b)
            return _
        lax.fori_loop(0, nloop, loop, None)

        wait_idx(1)
        wait_gather(0)
        start_gather(1)
        scatter(0)
        wait_gather(1)
        scatter(1)

        plsc.subcore_barrier()
        pltpu.sync_copy(shared.at[pl.ds(row0, RPT)],
                        out_hbm.at[c, pl.ds(row0, RPT)])

    return pl.kernel(
        body,
        out_type=jax.ShapeDtypeStruct((2, NP, H), jnp.float32),
        mesh=_mesh(),
        scratch_types=[
            pltpu.VMEM((2, CH), jnp.int32),
            pltpu.VMEM((2, CH), jnp.int32),
            pltpu.VMEM((2, CH, H), jnp.float32),
            pltpu.VMEM_SHARED((NP, H), jnp.float32),
            pltpu.SemaphoreType.DMA,
            pltpu.SemaphoreType.DMA,
            pltpu.SemaphoreType.DMA,
            pltpu.SemaphoreType.DMA,
        ],
    )(g, src2, dst2)


def kernel(x_p_id, edge_index_p, x_p_batch, emb0, conv_W, conv_b,
           lin0_W, lin0_b, bn_gamma, bn_beta, lin1_W, lin1_b):
    f32 = jnp.float32
    ids3 = jnp.pad(x_p_id.astype(jnp.int32), (0, NP - N)).reshape(NP // 128, 1, 128)
    embp = jnp.pad(emb0, ((0, VP - VOCAB), (0, 0)))
    src2 = jnp.pad(edge_index_p[0].astype(jnp.int32), (0, EP - E),
                   constant_values=N).reshape(NTILES * KCH, CH)
    dst2 = jnp.pad(edge_index_p[1].astype(jnp.int32), (0, EP - E),
                   constant_values=N).reshape(NTILES * KCH, CH)
    dst_deg = jnp.pad(edge_index_p[1].astype(jnp.int32), (0, EP - E),
                      constant_values=N).reshape(EP // 2048, 1, 2048)
    bat3 = jnp.pad(x_p_batch.astype(jnp.int32), (0, NP - N),
                   constant_values=B + 1).reshape(NP // RB, 1, RB)
    b2 = [conv_b[i].reshape(1, H) for i in range(L)]
    l0b = lin0_b.reshape(1, H)
    gam = bn_gamma.reshape(1, H)
    bet = bn_beta.reshape(1, H)
    l1wp = jnp.pad(lin1_W, ((0, 0), (0, H - 2)))
    l1bp = jnp.pad(lin1_b, (0, H - 2), constant_values=-1e30).reshape(1, H)

    x0 = _embed(ids3, embp)
    deg2 = _deg(dst_deg)
    dinvH = _dinv(deg2.T.reshape(NP, 1))

    g = _mm0(x0, dinvH, conv_W[0])
    parts = _msg_sc(g, src2, dst2)
    for i in (1, 2):
        g = _mmf(parts, g, dinvH, b2[i - 1], conv_W[i])
        parts = _msg_sc(g, src2, dst2)

    out = _pool(parts, g, dinvH, b2[2], bat3, lin0_W, l0b, gam, bet,
                l1wp, l1bp)
    return out[:, :2].astype(f32)
